# baseline (speedup 1.0000x reference)
import jax
import jax.numpy as jnp
from jax.experimental import pallas as pl
from jax.experimental.pallas import tpu as pltpu
from jax.sharding import Mesh, PartitionSpec as P

try:
    from jax import shard_map as _shard_map
except ImportError:
    from jax.experimental.shard_map import shard_map as _shard_map

_N = 10000
_BM1 = 200
_BMR = 1000
_F8 = jnp.float8_e4m3fn


def _fp8_scale(sv):
    mx = jnp.max(jnp.abs(sv), axis=0, keepdims=True)
    mx = jnp.max(mx, axis=1, keepdims=True)
    bits = jax.lax.bitcast_convert_type(mx, jnp.int32)
    ebits = jnp.clip((bits >> 23) & 0xFF, 10, 250)
    alpha = jax.lax.bitcast_convert_type((261 - ebits) << 23, jnp.float32)
    dealpha = jax.lax.bitcast_convert_type((ebits - 7) << 23, jnp.float32)
    return alpha, dealpha


def _head_body(x_ref, w1t_ref, b1_ref, wg1_ref, s1_ref, cs1_ref):
    h = jnp.dot(x_ref[...], w1t_ref[...], preferred_element_type=jnp.float32)
    h = h + b1_ref[...]
    s1 = jnp.dot(h, wg1_ref[...], preferred_element_type=jnp.float32)
    s1_ref[...] = s1
    cs1_ref[...] = 0.5 * jnp.sum(s1, axis=0, keepdims=True)


def _l1_body(adj_ref, s_ref, cs_ref, b_ref, w_ref, adjq_ref, s2_ref, cs2_ref,
             s8_ref, de_ref):
    m = pl.program_id(0)

    @pl.when(m == 0)
    def _():
        sv = s_ref[...]
        alpha, dealpha = _fp8_scale(sv)
        s8_ref[...] = (sv * alpha).astype(_F8)
        de_ref[...] = dealpha

    ab8 = (adj_ref[...] - 0.5).astype(_F8)
    adjq_ref[...] = ab8
    agg = jnp.dot(ab8, s8_ref[...], preferred_element_type=jnp.float32)
    h = jnp.maximum(agg * de_ref[...] + cs_ref[...] + b_ref[...], 0.0)
    s2 = jnp.dot(h, w_ref[...], preferred_element_type=jnp.float32)
    s2_ref[...] = s2
    part = 0.5 * jnp.sum(s2, axis=0, keepdims=True)

    @pl.when(m == 0)
    def _():
        cs2_ref[...] = part

    @pl.when(m != 0)
    def _():
        cs2_ref[...] += part


def _mid_body(adjq_ref, s_ref, cs_ref, b_ref, w_ref, snext_ref, csn_ref,
              s8_ref, de_ref):
    m = pl.program_id(0)

    @pl.when(m == 0)
    def _():
        sv = s_ref[...]
        alpha, dealpha = _fp8_scale(sv)
        s8_ref[...] = (sv * alpha).astype(_F8)
        de_ref[...] = dealpha

    agg = jnp.dot(adjq_ref[...], s8_ref[...], preferred_element_type=jnp.float32)
    h = jnp.maximum(agg * de_ref[...] + cs_ref[...] + b_ref[...], 0.0)
    sn = jnp.dot(h, w_ref[...], preferred_element_type=jnp.float32)
    snext_ref[...] = sn
    part = 0.5 * jnp.sum(sn, axis=0, keepdims=True)

    @pl.when(m == 0)
    def _():
        csn_ref[...] = part

    @pl.when(m != 0)
    def _():
        csn_ref[...] += part


def _last_body(adjq_ref, s_ref, cs_ref, b_ref, wf2_ref, bf2_ref, out_ref,
               s8_ref, de_ref):
    m = pl.program_id(0)

    @pl.when(m == 0)
    def _():
        sv = s_ref[...]
        alpha, dealpha = _fp8_scale(sv)
        s8_ref[...] = (sv * alpha).astype(_F8)
        de_ref[...] = dealpha

    agg = jnp.dot(adjq_ref[...], s8_ref[...], preferred_element_type=jnp.float32)
    h = jnp.maximum(agg * de_ref[...] + cs_ref[...] + b_ref[...], 0.0)
    z = jnp.sum(h * wf2_ref[...], axis=1, keepdims=True) + bf2_ref[...]
    out_ref[...] = jax.nn.sigmoid(z)


def _impl(x, adj, W_fc1, b_fc1, W_gc1, b_gc1, W_gc2, b_gc2, W_gc3, b_gc3,
          W_gc4, b_gc4, W_fc2, b_fc2):
    f32 = jnp.float32
    m_loc = adj.shape[0]
    nmb1 = m_loc // _BM1
    nmbr = m_loc // _BMR

    s1, cs1 = pl.pallas_call(
        _head_body,
        out_shape=[
            jax.ShapeDtypeStruct((_N, 64), f32),
            jax.ShapeDtypeStruct((1, 64), f32),
        ],
    )(x, W_fc1.T, b_fc1.reshape(1, 128), W_gc1)

    adjq, s2l, cs2p = pl.pallas_call(
        _l1_body,
        grid=(nmb1,),
        in_specs=[
            pl.BlockSpec((_BM1, _N), lambda m: (m, 0)),
            pl.BlockSpec((_N, 64), lambda m: (0, 0)),
            pl.BlockSpec((1, 64), lambda m: (0, 0)),
            pl.BlockSpec((1, 64), lambda m: (0, 0)),
            pl.BlockSpec((64, 32), lambda m: (0, 0)),
        ],
        out_specs=[
            pl.BlockSpec((_BM1, _N), lambda m: (m, 0)),
            pl.BlockSpec((_BM1, 32), lambda m: (m, 0)),
            pl.BlockSpec((1, 32), lambda m: (0, 0)),
        ],
        out_shape=[
            jax.ShapeDtypeStruct((m_loc, _N), _F8),
            jax.ShapeDtypeStruct((m_loc, 32), f32),
            jax.ShapeDtypeStruct((1, 32), f32),
        ],
        scratch_shapes=[
            pltpu.VMEM((_N, 64), _F8),
            pltpu.VMEM((1, 1), f32),
        ],
    )(adj, s1, cs1, b_gc1.reshape(1, 64), W_gc2)

    s2 = jax.lax.all_gather(s2l, 'd', axis=0, tiled=True)
    cs2 = jax.lax.psum(cs2p, 'd')

    def _mid_pass(s, cs, b, w, dn):
        di = s.shape[1]
        return pl.pallas_call(
            _mid_body,
            grid=(nmbr,),
            in_specs=[
                pl.BlockSpec((_BMR, _N), lambda m: (m, 0)),
                pl.BlockSpec((_N, di), lambda m: (0, 0)),
                pl.BlockSpec((1, di), lambda m: (0, 0)),
                pl.BlockSpec((1, di), lambda m: (0, 0)),
                pl.BlockSpec((di, dn), lambda m: (0, 0)),
            ],
            out_specs=[
                pl.BlockSpec((_BMR, dn), lambda m: (m, 0)),
                pl.BlockSpec((1, dn), lambda m: (0, 0)),
            ],
            out_shape=[
                jax.ShapeDtypeStruct((m_loc, dn), f32),
                jax.ShapeDtypeStruct((1, dn), f32),
            ],
            scratch_shapes=[
                pltpu.VMEM((_N, di), _F8),
                pltpu.VMEM((1, 1), f32),
            ],
        )(adjq, s, cs, b, w)

    s3l, cs3p = _mid_pass(s2, cs2, b_gc2.reshape(1, 32), W_gc3, 32)
    s3 = jax.lax.all_gather(s3l, 'd', axis=0, tiled=True)
    cs3 = jax.lax.psum(cs3p, 'd')

    s4l, cs4p = _mid_pass(s3, cs3, b_gc3.reshape(1, 32), W_gc4, 16)
    s4 = jax.lax.all_gather(s4l, 'd', axis=0, tiled=True)
    cs4 = jax.lax.psum(cs4p, 'd')

    out = pl.pallas_call(
        _last_body,
        grid=(nmbr,),
        in_specs=[
            pl.BlockSpec((_BMR, _N), lambda m: (m, 0)),
            pl.BlockSpec((_N, 16), lambda m: (0, 0)),
            pl.BlockSpec((1, 16), lambda m: (0, 0)),
            pl.BlockSpec((1, 16), lambda m: (0, 0)),
            pl.BlockSpec((1, 16), lambda m: (0, 0)),
            pl.BlockSpec((1, 1), lambda m: (0, 0)),
        ],
        out_specs=pl.BlockSpec((_BMR, 1), lambda m: (m, 0)),
        out_shape=jax.ShapeDtypeStruct((m_loc, 1), f32),
        scratch_shapes=[
            pltpu.VMEM((_N, 16), _F8),
            pltpu.VMEM((1, 1), f32),
        ],
    )(adjq, s4, cs4, b_gc4.reshape(1, 16), W_fc2, b_fc2.reshape(1, 1))

    return out


def kernel(x, adj, W_fc1, b_fc1, W_gc1, b_gc1, W_gc2, b_gc2, W_gc3, b_gc3,
           W_gc4, b_gc4, W_fc2, b_fc2):
    devs = jax.devices()
    ndev = 2 if len(devs) >= 2 and _N % (2 * _BMR) == 0 else 1
    mesh = Mesh(devs[:ndev], ('d',))
    rep = P(None, None)
    spec_in = (rep, P('d', None)) + (rep,) * 12
    args = (x, adj, W_fc1, b_fc1.reshape(1, 128), W_gc1, b_gc1.reshape(1, 64),
            W_gc2, b_gc2.reshape(1, 32), W_gc3, b_gc3.reshape(1, 32),
            W_gc4, b_gc4.reshape(1, 16), W_fc2, b_fc2.reshape(1, 1))

    def wrapped(x, adj, W_fc1, b_fc1, W_gc1, b_gc1, W_gc2, b_gc2, W_gc3,
                b_gc3, W_gc4, b_gc4, W_fc2, b_fc2):
        return _impl(x, adj, W_fc1, b_fc1.reshape(-1), W_gc1,
                     b_gc1.reshape(-1), W_gc2, b_gc2.reshape(-1), W_gc3,
                     b_gc3.reshape(-1), W_gc4, b_gc4.reshape(-1), W_fc2,
                     b_fc2.reshape(-1))

    try:
        f = _shard_map(wrapped, mesh=mesh, in_specs=spec_in,
                       out_specs=P('d', None), check_vma=False)
    except TypeError:
        f = _shard_map(wrapped, mesh=mesh, in_specs=spec_in,
                       out_specs=P('d', None), check_rep=False)
    return f(*args)


# speedup vs baseline: 3.3495x; 3.3495x over previous
import jax
import jax.numpy as jnp
from jax.experimental import pallas as pl
from jax.experimental.pallas import tpu as pltpu

_N = 10000
_BM = 400
_BMR = 1000
_F8 = jnp.float8_e4m3fn


def _fp8_scale(sv):
    mx = jnp.max(jnp.abs(sv), axis=0, keepdims=True)
    mx = jnp.max(mx, axis=1, keepdims=True)
    bits = jax.lax.bitcast_convert_type(mx, jnp.int32)
    ebits = jnp.clip((bits >> 23) & 0xFF, 10, 250)
    alpha = jax.lax.bitcast_convert_type((261 - ebits) << 23, jnp.float32)
    dealpha = jax.lax.bitcast_convert_type((ebits - 7) << 23, jnp.float32)
    return alpha, dealpha


def _head_body(x_ref, w1t_ref, b1_ref, wg1_ref, s1_ref, cs1_ref):
    h = jnp.dot(x_ref[...], w1t_ref[...], preferred_element_type=jnp.float32)
    h = h + b1_ref[...]
    s1 = jnp.dot(h, wg1_ref[...], preferred_element_type=jnp.float32)
    s1_ref[...] = s1
    cs1_ref[...] = 0.5 * jnp.sum(s1, axis=0, keepdims=True)


def _l1_body(adj_ref, s_ref, cs_ref, b_ref, w_ref, adjq_ref, s2_ref, cs2_ref,
             s8_ref, de_ref):
    m = pl.program_id(0)

    @pl.when(m == 0)
    def _():
        sv = s_ref[...]
        alpha, dealpha = _fp8_scale(sv)
        s8_ref[...] = (sv * alpha).astype(_F8)
        de_ref[...] = dealpha

    ab8 = (adj_ref[...] - 0.5).astype(_F8)
    adjq_ref[...] = ab8
    agg = jnp.dot(ab8, s8_ref[...], preferred_element_type=jnp.float32)
    h = jnp.maximum(agg * de_ref[...] + cs_ref[...] + b_ref[...], 0.0)
    s2 = jnp.dot(h, w_ref[...], preferred_element_type=jnp.float32)
    s2_ref[...] = s2
    part = 0.5 * jnp.sum(s2, axis=0, keepdims=True)

    @pl.when(m == 0)
    def _():
        cs2_ref[...] = part

    @pl.when(m != 0)
    def _():
        cs2_ref[...] += part


def _tail_body(adjq_ref, s2_ref, cs2_ref, b2_ref, b3_ref, b4_ref,
               wg3_ref, wg4_ref, wf2_ref, bf2_ref, out_ref,
               s8a_ref, s8b_ref, s8c_ref, dea_ref, deb_ref, dec_ref,
               cs3_ref, cs4_ref, sn3_ref, sn4_ref):
    p = pl.program_id(0)
    l = p // 10
    mq = p % 10

    @pl.when(p == 0)
    def _():
        sv = s2_ref[...]
        alpha, dealpha = _fp8_scale(sv)
        s8a_ref[...] = (sv * alpha).astype(_F8)
        dea_ref[...] = dealpha

    @pl.when(l == 0)
    def _():
        agg = jnp.dot(adjq_ref[...], s8a_ref[...],
                      preferred_element_type=jnp.float32)
        h = jnp.maximum(agg * dea_ref[...] + cs2_ref[...] + b2_ref[...], 0.0)
        sn = jnp.dot(h, wg3_ref[...], preferred_element_type=jnp.float32)
        sn3_ref[pl.ds(mq * _BMR, _BMR), :] = sn
        part = 0.5 * jnp.sum(sn, axis=0, keepdims=True)

        @pl.when(mq == 0)
        def _():
            cs3_ref[...] = part

        @pl.when(mq != 0)
        def _():
            cs3_ref[...] += part

    @pl.when(p == 10)
    def _():
        sv = sn3_ref[...]
        alpha, dealpha = _fp8_scale(sv)
        s8b_ref[...] = (sv * alpha).astype(_F8)
        deb_ref[...] = dealpha

    @pl.when(l == 1)
    def _():
        agg = jnp.dot(adjq_ref[...], s8b_ref[...],
                      preferred_element_type=jnp.float32)
        h = jnp.maximum(agg * deb_ref[...] + cs3_ref[...] + b3_ref[...], 0.0)
        sn = jnp.dot(h, wg4_ref[...], preferred_element_type=jnp.float32)
        sn4_ref[pl.ds(mq * _BMR, _BMR), :] = sn
        part = 0.5 * jnp.sum(sn, axis=0, keepdims=True)

        @pl.when(mq == 0)
        def _():
            cs4_ref[...] = part

        @pl.when(mq != 0)
        def _():
            cs4_ref[...] += part

    @pl.when(p == 20)
    def _():
        sv = sn4_ref[...]
        alpha, dealpha = _fp8_scale(sv)
        s8c_ref[...] = (sv * alpha).astype(_F8)
        dec_ref[...] = dealpha

    @pl.when(l == 2)
    def _():
        agg = jnp.dot(adjq_ref[...], s8c_ref[...],
                      preferred_element_type=jnp.float32)
        h = jnp.maximum(agg * dec_ref[...] + cs4_ref[...] + b4_ref[...], 0.0)
        z = jnp.sum(h * wf2_ref[...], axis=1, keepdims=True) + bf2_ref[...]
        out_ref[...] = jax.nn.sigmoid(z)


def kernel(x, adj, W_fc1, b_fc1, W_gc1, b_gc1, W_gc2, b_gc2, W_gc3, b_gc3,
           W_gc4, b_gc4, W_fc2, b_fc2):
    f32 = jnp.float32
    nmb = _N // _BM

    s1, cs1 = pl.pallas_call(
        _head_body,
        out_shape=[
            jax.ShapeDtypeStruct((_N, 64), f32),
            jax.ShapeDtypeStruct((1, 64), f32),
        ],
    )(x, W_fc1.T, b_fc1.reshape(1, 128), W_gc1)

    adjq, s2, cs2 = pl.pallas_call(
        _l1_body,
        grid=(nmb,),
        in_specs=[
            pl.BlockSpec((_BM, _N), lambda m: (m, 0)),
            pl.BlockSpec((_N, 64), lambda m: (0, 0)),
            pl.BlockSpec((1, 64), lambda m: (0, 0)),
            pl.BlockSpec((1, 64), lambda m: (0, 0)),
            pl.BlockSpec((64, 32), lambda m: (0, 0)),
        ],
        out_specs=[
            pl.BlockSpec((_BM, _N), lambda m: (m, 0)),
            pl.BlockSpec((_BM, 32), lambda m: (m, 0)),
            pl.BlockSpec((1, 32), lambda m: (0, 0)),
        ],
        out_shape=[
            jax.ShapeDtypeStruct((_N, _N), _F8),
            jax.ShapeDtypeStruct((_N, 32), f32),
            jax.ShapeDtypeStruct((1, 32), f32),
        ],
        scratch_shapes=[
            pltpu.VMEM((_N, 64), _F8),
            pltpu.VMEM((1, 1), f32),
        ],
    )(adj, s1, cs1, b_gc1.reshape(1, 64), W_gc2)

    nmbr = _N // _BMR
    out = pl.pallas_call(
        _tail_body,
        grid=(3 * nmbr,),
        in_specs=[
            pl.BlockSpec((_BMR, _N), lambda p: (p % 10, 0)),
            pl.BlockSpec((_N, 32), lambda p: (0, 0)),
            pl.BlockSpec((1, 32), lambda p: (0, 0)),
            pl.BlockSpec((1, 32), lambda p: (0, 0)),
            pl.BlockSpec((1, 32), lambda p: (0, 0)),
            pl.BlockSpec((1, 16), lambda p: (0, 0)),
            pl.BlockSpec((32, 32), lambda p: (0, 0)),
            pl.BlockSpec((32, 16), lambda p: (0, 0)),
            pl.BlockSpec((1, 16), lambda p: (0, 0)),
            pl.BlockSpec((1, 1), lambda p: (0, 0)),
        ],
        out_specs=pl.BlockSpec((_BMR, 1), lambda p: (p % 10, 0)),
        out_shape=jax.ShapeDtypeStruct((_N, 1), f32),
        scratch_shapes=[
            pltpu.VMEM((_N, 32), _F8),
            pltpu.VMEM((_N, 32), _F8),
            pltpu.VMEM((_N, 16), _F8),
            pltpu.VMEM((1, 1), f32),
            pltpu.VMEM((1, 1), f32),
            pltpu.VMEM((1, 1), f32),
            pltpu.VMEM((1, 32), f32),
            pltpu.VMEM((1, 16), f32),
            pltpu.VMEM((_N, 32), f32),
            pltpu.VMEM((_N, 16), f32),
        ],
    )(adjq, s2, cs2, b_gc2.reshape(1, 32),
      b_gc3.reshape(1, 32), b_gc4.reshape(1, 16), W_gc3, W_gc4,
      W_fc2, b_fc2.reshape(1, 1))

    return out
